# Initial kernel scaffold; baseline (speedup 1.0000x reference)
import jax
import jax.numpy as jnp
from jax.experimental import pallas as pl


def kernel(xyz, new_xyz, rot, features):
    raise NotImplementedError("write your pallas kernel here")


# speedup vs baseline: 159.0347x; 159.0347x over previous
import functools

import jax
import jax.numpy as jnp
from jax import lax
from jax.experimental import pallas as pl
from jax.experimental.pallas import tpu as pltpu
from jax.experimental.pallas import tpu_sc as plsc

_RADIUS = 0.05
_HMIN = -0.02
_HMAX = 0.04
_S = 32
_N = 16384
_P = 512
_C = 128
_QB = 8
_CH = 2048
_NCH = _N // _CH
_SEP = 0.25
_NOVALID = 0.1
_BIGI = 2 ** 30

_f32 = jnp.float32
_bf = jnp.bfloat16


def _select_body(pts_ref, nq_ref, rot_ref, idx_ref, keys_ref):
    px = pts_ref[0:1, :]
    py = pts_ref[1:2, :]
    pz = pts_ref[2:3, :]
    qx = nq_ref[:, 0:1]
    qy = nq_ref[:, 1:2]
    qz = nq_ref[:, 2:3]

    qb = nq_ref[...].astype(_bf)
    pb = pts_ref[0:3, :].astype(_bf)
    dot = lax.dot_general(qb, pb, (((1,), (0,)), ((), ())),
                          preferred_element_type=_f32)
    q2 = (qx * qx + qy * qy) + qz * qz
    p2 = (px * px + py * py) + pz * pz
    d2 = q2 + p2 - 2.0 * dot

    dx = px - qx
    dy = py - qy
    dz = pz - qz
    xcl, ycl, zcl = [], [], []
    for q in range(_QB):
        deltaT = jnp.concatenate([dx[q:q + 1], dy[q:q + 1], dz[q:q + 1]],
                                 axis=0)
        R3 = jnp.concatenate([rot_ref[q:q + 1, 0:3], rot_ref[q:q + 1, 3:6],
                              rot_ref[q:q + 1, 6:9]], axis=0)
        cyl = lax.dot_general(R3.astype(_bf), deltaT.astype(_bf),
                              (((1,), (0,)), ((), ())),
                              preferred_element_type=_f32)
        xcl.append(cyl[0:1])
        ycl.append(cyl[1:2])
        zcl.append(cyl[2:3])
    xc = jnp.concatenate(xcl, axis=0)
    yc = jnp.concatenate(ycl, axis=0)
    zc = jnp.concatenate(zcl, axis=0)

    rad = jnp.sqrt(yc * yc + zc * zc)
    sc = rad + 0.001 * jnp.abs(xc)
    in_cyl = (rad <= _RADIUS) & (xc >= _HMIN) & (xc <= _HMAX)
    keys_ref[...] = jnp.where(in_cyl, sc, _SEP + d2)

    col = lax.broadcasted_iota(jnp.int32, (_QB, _S), 1)
    inf = jnp.full((_QB, 1), jnp.inf, _f32)

    def round_fn(t, carry):
        acc, nprev, k0 = carry
        m = inf
        for c in range(_NCH):
            sl = pl.ds(c * _CH, _CH)
            v = keys_ref[:, sl]
            li = lax.broadcasted_iota(jnp.int32, (_QB, _CH), 1) + c * _CH
            v = jnp.where(li == nprev, jnp.inf, v)
            keys_ref[:, sl] = v
            m = jnp.minimum(m, jnp.min(v, axis=1, keepdims=True))
        n = jnp.full((_QB, 1), _BIGI, jnp.int32)
        for c in range(_NCH):
            sl = pl.ds(c * _CH, _CH)
            v = keys_ref[:, sl]
            li = lax.broadcasted_iota(jnp.int32, (_QB, _CH), 1) + c * _CH
            cand = jnp.where(v == m, li, _BIGI)
            n = jnp.minimum(n, jnp.min(cand, axis=1, keepdims=True))
        acc = jnp.where(col == t, jnp.broadcast_to(n, (_QB, _S)), acc)
        k0 = jnp.where(t == 0, m, k0)
        return acc, n, k0

    acc0 = jnp.zeros((_QB, _S), jnp.int32)
    nprev0 = jnp.full((_QB, 1), -1, jnp.int32)
    acc, _, k0 = lax.fori_loop(0, _S, round_fn, (acc0, nprev0, inf))

    no_valid = k0 >= _NOVALID
    idx_ref[...] = jnp.where(no_valid, jnp.broadcast_to(acc[:, 0:1], (_QB, _S)),
                             acc)


def _tc_select(pts, nq, rot9):
    return pl.pallas_call(
        _select_body,
        grid=(_P // _QB,),
        in_specs=[
            pl.BlockSpec((8, _N), lambda i: (0, 0)),
            pl.BlockSpec((_QB, 3), lambda i: (i, 0)),
            pl.BlockSpec((_QB, 9), lambda i: (i, 0)),
        ],
        out_specs=pl.BlockSpec((_QB, _S), lambda i: (i, 0)),
        out_shape=jax.ShapeDtypeStruct((_P, _S), jnp.int32),
        scratch_shapes=[pltpu.VMEM((_QB, _N), _f32)],
    )(pts, nq, rot9)


_NW = 32
_BW = (_P * _S) // _NW
_QW = _BW // _S


def _sc_body(featT, xyzw, idx2, qr, outF, outX,
             idx_v, rows_v, q_v, ox_v, sem):
    wid = lax.axis_index("s") * 2 + lax.axis_index("c")
    base = wid * _BW

    pltpu.sync_copy(idx2.at[pl.ds(wid * 4, 4)], idx_v)
    pltpu.sync_copy(qr.at[pl.ds(wid * _QW, _QW)], q_v)

    copies = []
    for j in range(4):
        copies.append(pltpu.async_copy(
            featT.at[idx_v.at[j]], rows_v.at[pl.ds(j * 128, 128)], sem))
    for c in copies:
        c.wait()
    pltpu.sync_copy(rows_v, outF.at[pl.ds(base, _BW)])

    copies = []
    for j in range(4):
        copies.append(pltpu.async_copy(
            xyzw.at[idx_v.at[j]], rows_v.at[pl.ds(j * 128, 128)], sem))
    for c in copies:
        c.wait()

    li = lax.iota(jnp.int32, 16)
    for pq in range(_QW):
        qrow = q_v[pq, pl.ds(0, 16)]
        cx = qrow[0]
        cy = qrow[1]
        cz = qrow[2]
        r = [qrow[3 + k] for k in range(9)]
        for h in range(2):
            off = pq * _S + h * 16
            gx = jnp.zeros((16,), _f32)
            gy = jnp.zeros((16,), _f32)
            gz = jnp.zeros((16,), _f32)
            for j in range(16):
                row = rows_v[off + j, pl.ds(0, 16)]
                gx = jnp.where(li == j, row[0], gx)
                gy = jnp.where(li == j, row[1], gy)
                gz = jnp.where(li == j, row[2], gz)
            dxb = gx - cx
            dyb = gy - cy
            dzb = gz - cz
            ox_v[0, pl.ds(off, 16)] = (dxb * r[0] + dyb * r[3]) + dzb * r[6]
            ox_v[1, pl.ds(off, 16)] = (dxb * r[1] + dyb * r[4]) + dzb * r[7]
            ox_v[2, pl.ds(off, 16)] = (dxb * r[2] + dyb * r[5]) + dzb * r[8]

    pltpu.sync_copy(ox_v, outX.at[:, pl.ds(base, _BW)])


def _sc_group(featT, xyzw, idx2, qr):
    mesh = plsc.VectorSubcoreMesh(core_axis_name="c", subcore_axis_name="s")
    fn = functools.partial(
        pl.kernel,
        mesh=mesh,
        out_type=[
            jax.ShapeDtypeStruct((_P * _S, _C), _f32),
            jax.ShapeDtypeStruct((3, _P * _S), _f32),
        ],
        scratch_types=[
            pltpu.VMEM((4, 128), jnp.int32),
            pltpu.VMEM((_BW, _C), _f32),
            pltpu.VMEM((_QW, 16), _f32),
            pltpu.VMEM((3, _BW), _f32),
            pltpu.SemaphoreType.DMA,
        ],
    )(_sc_body)
    return fn(featT, xyzw, idx2, qr)


def kernel(xyz, new_xyz, rot, features):
    B = xyz.shape[0]
    xyz0 = xyz[0]
    nq = new_xyz[0]
    rot9 = rot[0].reshape(_P, 9)

    pts = jnp.zeros((8, _N), _f32).at[0:3, :].set(jnp.swapaxes(xyz0, 0, 1))
    idx = _tc_select(pts, nq, rot9)

    featT = jnp.swapaxes(features[0], 0, 1)
    idx2 = idx.reshape(128, 128)
    qr = jnp.zeros((_P, 16), _f32)
    qr = qr.at[:, 0:3].set(nq).at[:, 3:12].set(rot9)
    xyzw = jnp.zeros((_N, 128), _f32).at[:, 0:3].set(xyz0)

    outF, outX = _sc_group(featT, xyzw, idx2, qr)
    gx = outX.reshape(1, 3, _P, _S)
    gf = jnp.swapaxes(outF, 0, 1).reshape(1, _C, _P, _S)
    return jnp.concatenate([gx, gf], axis=1)


# speedup vs baseline: 173.3865x; 1.0902x over previous
import functools

import jax
import jax.numpy as jnp
from jax import lax
from jax.experimental import pallas as pl
from jax.experimental.pallas import tpu as pltpu
from jax.experimental.pallas import tpu_sc as plsc

_RADIUS = 0.05
_HMIN = -0.02
_HMAX = 0.04
_S = 32
_N = 16384
_P = 512
_C = 128
_QB = 8
_CH = 2048
_NCH = _N // _CH
_SEP = 0.25
_NOVALID = 0.1
_BIGI = 2 ** 30

_f32 = jnp.float32
_bf = jnp.bfloat16


def _select_body(pts_ref, nq_ref, rot_ref, idx_ref, keys_ref):
    px = pts_ref[0:1, :]
    py = pts_ref[1:2, :]
    pz = pts_ref[2:3, :]
    qx = nq_ref[:, 0:1]
    qy = nq_ref[:, 1:2]
    qz = nq_ref[:, 2:3]

    qb = nq_ref[...].astype(_bf)
    pb = pts_ref[0:3, :].astype(_bf)
    dot = lax.dot_general(qb, pb, (((1,), (0,)), ((), ())),
                          preferred_element_type=_f32)
    q2 = (qx * qx + qy * qy) + qz * qz
    p2 = (px * px + py * py) + pz * pz
    d2 = q2 + p2 - 2.0 * dot

    dx = px - qx
    dy = py - qy
    dz = pz - qz
    xcl, ycl, zcl = [], [], []
    for q in range(_QB):
        deltaT = jnp.concatenate([dx[q:q + 1], dy[q:q + 1], dz[q:q + 1]],
                                 axis=0)
        R3 = jnp.concatenate([rot_ref[q:q + 1, 0:3], rot_ref[q:q + 1, 3:6],
                              rot_ref[q:q + 1, 6:9]], axis=0)
        cyl = lax.dot_general(R3.astype(_bf), deltaT.astype(_bf),
                              (((1,), (0,)), ((), ())),
                              preferred_element_type=_f32)
        xcl.append(cyl[0:1])
        ycl.append(cyl[1:2])
        zcl.append(cyl[2:3])
    xc = jnp.concatenate(xcl, axis=0)
    yc = jnp.concatenate(ycl, axis=0)
    zc = jnp.concatenate(zcl, axis=0)

    rad = jnp.sqrt(yc * yc + zc * zc)
    sc = rad + 0.001 * jnp.abs(xc)
    in_cyl = (rad <= _RADIUS) & (xc >= _HMIN) & (xc <= _HMAX)
    keys_ref[...] = jnp.where(in_cyl, sc, _SEP + d2)

    col = lax.broadcasted_iota(jnp.int32, (_QB, _S), 1)
    inf = jnp.full((_QB, 1), jnp.inf, _f32)

    lidx = lax.broadcasted_iota(jnp.int32, (_QB, _CH), 1)

    def round_fn(t, carry):
        acc, nprev, k0 = carry
        vmin = jnp.full((_QB, _CH), jnp.inf, _f32)
        cidx = jnp.zeros((_QB, _CH), jnp.int32)
        for c in range(_NCH):
            sl = pl.ds(c * _CH, _CH)
            v = keys_ref[:, sl]
            v = jnp.where(lidx + c * _CH == nprev, jnp.inf, v)
            keys_ref[:, sl] = v
            newmin = v < vmin
            cidx = jnp.where(newmin, c, cidx)
            vmin = jnp.minimum(vmin, v)
        m = jnp.min(vmin, axis=1, keepdims=True)
        n_full = cidx * _CH + lidx
        cand = jnp.where(vmin == m, n_full, _BIGI)
        n = jnp.min(cand, axis=1, keepdims=True)
        acc = jnp.where(col == t, jnp.broadcast_to(n, (_QB, _S)), acc)
        k0 = jnp.where(t == 0, m, k0)
        return acc, n, k0

    acc0 = jnp.zeros((_QB, _S), jnp.int32)
    nprev0 = jnp.full((_QB, 1), -1, jnp.int32)
    acc, _, k0 = lax.fori_loop(0, _S, round_fn, (acc0, nprev0, inf))

    no_valid = k0 >= _NOVALID
    idx_ref[...] = jnp.where(no_valid, jnp.broadcast_to(acc[:, 0:1], (_QB, _S)),
                             acc)


def _tc_select(pts, nq, rot9):
    return pl.pallas_call(
        _select_body,
        grid=(_P // _QB,),
        in_specs=[
            pl.BlockSpec((8, _N), lambda i: (0, 0)),
            pl.BlockSpec((_QB, 3), lambda i: (i, 0)),
            pl.BlockSpec((_QB, 9), lambda i: (i, 0)),
        ],
        out_specs=pl.BlockSpec((_QB, _S), lambda i: (i, 0)),
        out_shape=jax.ShapeDtypeStruct((_P, _S), jnp.int32),
        scratch_shapes=[pltpu.VMEM((_QB, _N), _f32)],
    )(pts, nq, rot9)


_NW = 32
_BW = (_P * _S) // _NW
_QW = _BW // _S


def _sc_body(featT, xyzw, idx2, qr, outF, outX,
             idx_v, rows_v, q_v, ox_v, sem):
    wid = lax.axis_index("s") * 2 + lax.axis_index("c")
    base = wid * _BW

    pltpu.sync_copy(idx2.at[pl.ds(wid * 4, 4)], idx_v)
    pltpu.sync_copy(qr.at[pl.ds(wid * _QW, _QW)], q_v)

    copies = []
    for j in range(4):
        copies.append(pltpu.async_copy(
            featT.at[idx_v.at[j]], rows_v.at[pl.ds(j * 128, 128)], sem))
    for c in copies:
        c.wait()
    pltpu.sync_copy(rows_v, outF.at[pl.ds(base, _BW)])

    copies = []
    for j in range(4):
        copies.append(pltpu.async_copy(
            xyzw.at[idx_v.at[j]], rows_v.at[pl.ds(j * 128, 128)], sem))
    for c in copies:
        c.wait()

    li = lax.iota(jnp.int32, 16)
    for pq in range(_QW):
        qrow = q_v[pq, pl.ds(0, 16)]
        cx = qrow[0]
        cy = qrow[1]
        cz = qrow[2]
        r = [qrow[3 + k] for k in range(9)]
        for h in range(2):
            off = pq * _S + h * 16
            gx = jnp.zeros((16,), _f32)
            gy = jnp.zeros((16,), _f32)
            gz = jnp.zeros((16,), _f32)
            for j in range(16):
                row = rows_v[off + j, pl.ds(0, 16)]
                gx = jnp.where(li == j, row[0], gx)
                gy = jnp.where(li == j, row[1], gy)
                gz = jnp.where(li == j, row[2], gz)
            dxb = gx - cx
            dyb = gy - cy
            dzb = gz - cz
            ox_v[0, pl.ds(off, 16)] = (dxb * r[0] + dyb * r[3]) + dzb * r[6]
            ox_v[1, pl.ds(off, 16)] = (dxb * r[1] + dyb * r[4]) + dzb * r[7]
            ox_v[2, pl.ds(off, 16)] = (dxb * r[2] + dyb * r[5]) + dzb * r[8]

    pltpu.sync_copy(ox_v, outX.at[:, pl.ds(base, _BW)])


def _sc_group(featT, xyzw, idx2, qr):
    mesh = plsc.VectorSubcoreMesh(core_axis_name="c", subcore_axis_name="s")
    fn = functools.partial(
        pl.kernel,
        mesh=mesh,
        out_type=[
            jax.ShapeDtypeStruct((_P * _S, _C), _f32),
            jax.ShapeDtypeStruct((3, _P * _S), _f32),
        ],
        scratch_types=[
            pltpu.VMEM((4, 128), jnp.int32),
            pltpu.VMEM((_BW, _C), _f32),
            pltpu.VMEM((_QW, 16), _f32),
            pltpu.VMEM((3, _BW), _f32),
            pltpu.SemaphoreType.DMA,
        ],
    )(_sc_body)
    return fn(featT, xyzw, idx2, qr)


def kernel(xyz, new_xyz, rot, features):
    B = xyz.shape[0]
    xyz0 = xyz[0]
    nq = new_xyz[0]
    rot9 = rot[0].reshape(_P, 9)

    pts = jnp.zeros((8, _N), _f32).at[0:3, :].set(jnp.swapaxes(xyz0, 0, 1))
    idx = _tc_select(pts, nq, rot9)

    featT = jnp.swapaxes(features[0], 0, 1)
    idx2 = idx.reshape(128, 128)
    qr = jnp.zeros((_P, 16), _f32)
    qr = qr.at[:, 0:3].set(nq).at[:, 3:12].set(rot9)
    xyzw = jnp.zeros((_N, 128), _f32).at[:, 0:3].set(xyz0)

    outF, outX = _sc_group(featT, xyzw, idx2, qr)
    gx = outX.reshape(1, 3, _P, _S)
    gf = jnp.swapaxes(outF, 0, 1).reshape(1, _C, _P, _S)
    return jnp.concatenate([gx, gf], axis=1)


# speedup vs baseline: 177.3310x; 1.0227x over previous
import functools

import jax
import jax.numpy as jnp
from jax import lax
from jax.experimental import pallas as pl
from jax.experimental.pallas import tpu as pltpu
from jax.experimental.pallas import tpu_sc as plsc

_RADIUS = 0.05
_HMIN = -0.02
_HMAX = 0.04
_S = 32
_N = 16384
_P = 512
_C = 128
_QB = 8
_CH = 1024
_NCH = _N // _CH
_SEP = 0.25
_NOVALID = 0.1
_BIGI = 2 ** 30

_f32 = jnp.float32
_bf = jnp.bfloat16


def _select_body(pts_ref, nq_ref, rot_ref, idx_ref, keys_ref):
    px = pts_ref[0:1, :]
    py = pts_ref[1:2, :]
    pz = pts_ref[2:3, :]
    qx = nq_ref[:, 0:1]
    qy = nq_ref[:, 1:2]
    qz = nq_ref[:, 2:3]

    qb = nq_ref[...].astype(_bf)
    pb = pts_ref[0:3, :].astype(_bf)
    dot = lax.dot_general(qb, pb, (((1,), (0,)), ((), ())),
                          preferred_element_type=_f32)
    q2 = (qx * qx + qy * qy) + qz * qz
    p2 = (px * px + py * py) + pz * pz
    d2 = q2 + p2 - 2.0 * dot

    dx = px - qx
    dy = py - qy
    dz = pz - qz
    xcl, ycl, zcl = [], [], []
    for q in range(_QB):
        deltaT = jnp.concatenate([dx[q:q + 1], dy[q:q + 1], dz[q:q + 1]],
                                 axis=0)
        R3 = jnp.concatenate([rot_ref[q:q + 1, 0:3], rot_ref[q:q + 1, 3:6],
                              rot_ref[q:q + 1, 6:9]], axis=0)
        cyl = lax.dot_general(R3.astype(_bf), deltaT.astype(_bf),
                              (((1,), (0,)), ((), ())),
                              preferred_element_type=_f32)
        xcl.append(cyl[0:1])
        ycl.append(cyl[1:2])
        zcl.append(cyl[2:3])
    xc = jnp.concatenate(xcl, axis=0)
    yc = jnp.concatenate(ycl, axis=0)
    zc = jnp.concatenate(zcl, axis=0)

    rad = jnp.sqrt(yc * yc + zc * zc)
    sc = rad + 0.001 * jnp.abs(xc)
    in_cyl = (rad <= _RADIUS) & (xc >= _HMIN) & (xc <= _HMAX)
    keys_ref[...] = jnp.where(in_cyl, sc, _SEP + d2)

    col = lax.broadcasted_iota(jnp.int32, (_QB, _S), 1)
    inf = jnp.full((_QB, 1), jnp.inf, _f32)

    lidx = lax.broadcasted_iota(jnp.int32, (_QB, _CH), 1)

    def round_fn(t, carry):
        acc, nprev, k0 = carry
        vmin = jnp.full((_QB, _CH), jnp.inf, _f32)
        cidx = jnp.zeros((_QB, _CH), jnp.int32)
        for c in range(_NCH):
            sl = pl.ds(c * _CH, _CH)
            v = keys_ref[:, sl]
            v = jnp.where(lidx + c * _CH == nprev, jnp.inf, v)
            keys_ref[:, sl] = v
            newmin = v < vmin
            cidx = jnp.where(newmin, c, cidx)
            vmin = jnp.minimum(vmin, v)
        m = jnp.min(vmin, axis=1, keepdims=True)
        n_full = cidx * _CH + lidx
        cand = jnp.where(vmin == m, n_full, _BIGI)
        n = jnp.min(cand, axis=1, keepdims=True)
        acc = jnp.where(col == t, jnp.broadcast_to(n, (_QB, _S)), acc)
        k0 = jnp.where(t == 0, m, k0)
        return acc, n, k0

    acc0 = jnp.zeros((_QB, _S), jnp.int32)
    nprev0 = jnp.full((_QB, 1), -1, jnp.int32)
    acc, _, k0 = lax.fori_loop(0, _S, round_fn, (acc0, nprev0, inf))

    no_valid = k0 >= _NOVALID
    idx_ref[...] = jnp.where(no_valid, jnp.broadcast_to(acc[:, 0:1], (_QB, _S)),
                             acc)


def _tc_select(pts, nq, rot9):
    return pl.pallas_call(
        _select_body,
        grid=(_P // _QB,),
        in_specs=[
            pl.BlockSpec((8, _N), lambda i: (0, 0)),
            pl.BlockSpec((_QB, 3), lambda i: (i, 0)),
            pl.BlockSpec((_QB, 9), lambda i: (i, 0)),
        ],
        out_specs=pl.BlockSpec((_QB, _S), lambda i: (i, 0)),
        out_shape=jax.ShapeDtypeStruct((_P, _S), jnp.int32),
        scratch_shapes=[pltpu.VMEM((_QB, _N), _f32)],
    )(pts, nq, rot9)


_NW = 32
_BW = (_P * _S) // _NW
_QW = _BW // _S


def _sc_body(featT, xyzw, idx2, qr, outF, outX,
             idx_v, rows_v, q_v, ox_v, sem):
    wid = lax.axis_index("s") * 2 + lax.axis_index("c")
    base = wid * _BW

    pltpu.sync_copy(idx2.at[pl.ds(wid * 4, 4)], idx_v)
    pltpu.sync_copy(qr.at[pl.ds(wid * _QW, _QW)], q_v)

    copies = []
    for j in range(4):
        copies.append(pltpu.async_copy(
            featT.at[idx_v.at[j]], rows_v.at[pl.ds(j * 128, 128)], sem))
    for c in copies:
        c.wait()
    pltpu.sync_copy(rows_v, outF.at[pl.ds(base, _BW)])

    copies = []
    for j in range(4):
        copies.append(pltpu.async_copy(
            xyzw.at[idx_v.at[j]], rows_v.at[pl.ds(j * 128, 128)], sem))
    for c in copies:
        c.wait()

    li = lax.iota(jnp.int32, 16)
    for pq in range(_QW):
        qrow = q_v[pq, pl.ds(0, 16)]
        cx = qrow[0]
        cy = qrow[1]
        cz = qrow[2]
        r = [qrow[3 + k] for k in range(9)]
        for h in range(2):
            off = pq * _S + h * 16
            gx = jnp.zeros((16,), _f32)
            gy = jnp.zeros((16,), _f32)
            gz = jnp.zeros((16,), _f32)
            for j in range(16):
                row = rows_v[off + j, pl.ds(0, 16)]
                gx = jnp.where(li == j, row[0], gx)
                gy = jnp.where(li == j, row[1], gy)
                gz = jnp.where(li == j, row[2], gz)
            dxb = gx - cx
            dyb = gy - cy
            dzb = gz - cz
            ox_v[0, pl.ds(off, 16)] = (dxb * r[0] + dyb * r[3]) + dzb * r[6]
            ox_v[1, pl.ds(off, 16)] = (dxb * r[1] + dyb * r[4]) + dzb * r[7]
            ox_v[2, pl.ds(off, 16)] = (dxb * r[2] + dyb * r[5]) + dzb * r[8]

    pltpu.sync_copy(ox_v, outX.at[:, pl.ds(base, _BW)])


def _sc_group(featT, xyzw, idx2, qr):
    mesh = plsc.VectorSubcoreMesh(core_axis_name="c", subcore_axis_name="s")
    fn = functools.partial(
        pl.kernel,
        mesh=mesh,
        out_type=[
            jax.ShapeDtypeStruct((_P * _S, _C), _f32),
            jax.ShapeDtypeStruct((3, _P * _S), _f32),
        ],
        scratch_types=[
            pltpu.VMEM((4, 128), jnp.int32),
            pltpu.VMEM((_BW, _C), _f32),
            pltpu.VMEM((_QW, 16), _f32),
            pltpu.VMEM((3, _BW), _f32),
            pltpu.SemaphoreType.DMA,
        ],
    )(_sc_body)
    return fn(featT, xyzw, idx2, qr)


def kernel(xyz, new_xyz, rot, features):
    B = xyz.shape[0]
    xyz0 = xyz[0]
    nq = new_xyz[0]
    rot9 = rot[0].reshape(_P, 9)

    pts = jnp.zeros((8, _N), _f32).at[0:3, :].set(jnp.swapaxes(xyz0, 0, 1))
    idx = _tc_select(pts, nq, rot9)

    featT = jnp.swapaxes(features[0], 0, 1)
    idx2 = idx.reshape(128, 128)
    qr = jnp.zeros((_P, 16), _f32)
    qr = qr.at[:, 0:3].set(nq).at[:, 3:12].set(rot9)
    xyzw = jnp.zeros((_N, 128), _f32).at[:, 0:3].set(xyz0)

    outF, outX = _sc_group(featT, xyzw, idx2, qr)
    gx = outX.reshape(1, 3, _P, _S)
    gf = jnp.swapaxes(outF, 0, 1).reshape(1, _C, _P, _S)
    return jnp.concatenate([gx, gf], axis=1)


# speedup vs baseline: 177.4912x; 1.0009x over previous
import functools

import jax
import jax.numpy as jnp
from jax import lax
from jax.experimental import pallas as pl
from jax.experimental.pallas import tpu as pltpu
from jax.experimental.pallas import tpu_sc as plsc

_RADIUS = 0.05
_HMIN = -0.02
_HMAX = 0.04
_S = 32
_N = 16384
_P = 512
_C = 128
_QB = 8
_CH = 1024
_NCH = _N // _CH
_SEP = 0.25
_NOVALID = 0.1
_BIGI = 2 ** 30

_f32 = jnp.float32
_bf = jnp.bfloat16


def _select_body(pts_ref, nq_ref, rot_ref, idx_ref, keys_ref):
    px = pts_ref[0:1, :]
    py = pts_ref[1:2, :]
    pz = pts_ref[2:3, :]
    qx = nq_ref[:, 0:1]
    qy = nq_ref[:, 1:2]
    qz = nq_ref[:, 2:3]

    qb = nq_ref[...].astype(_bf)
    pb = pts_ref[0:3, :].astype(_bf)
    dot = lax.dot_general(qb, pb, (((1,), (0,)), ((), ())),
                          preferred_element_type=_f32)
    q2 = (qx * qx + qy * qy) + qz * qz
    p2 = (px * px + py * py) + pz * pz
    d2 = q2 + p2 - 2.0 * dot

    dx = px - qx
    dy = py - qy
    dz = pz - qz
    xcl, ycl, zcl = [], [], []
    for q in range(_QB):
        deltaT = jnp.concatenate([dx[q:q + 1], dy[q:q + 1], dz[q:q + 1]],
                                 axis=0)
        R3 = jnp.concatenate([rot_ref[q:q + 1, 0:3], rot_ref[q:q + 1, 3:6],
                              rot_ref[q:q + 1, 6:9]], axis=0)
        cyl = lax.dot_general(R3.astype(_bf), deltaT.astype(_bf),
                              (((1,), (0,)), ((), ())),
                              preferred_element_type=_f32)
        xcl.append(cyl[0:1])
        ycl.append(cyl[1:2])
        zcl.append(cyl[2:3])
    xc = jnp.concatenate(xcl, axis=0)
    yc = jnp.concatenate(ycl, axis=0)
    zc = jnp.concatenate(zcl, axis=0)

    rad = jnp.sqrt(yc * yc + zc * zc)
    sc = rad + 0.001 * jnp.abs(xc)
    in_cyl = (rad <= _RADIUS) & (xc >= _HMIN) & (xc <= _HMAX)
    keys_ref[...] = jnp.where(in_cyl, sc, _SEP + d2)

    col = lax.broadcasted_iota(jnp.int32, (_QB, _S), 1)
    inf = jnp.full((_QB, 1), jnp.inf, _f32)

    lidx = lax.broadcasted_iota(jnp.int32, (_QB, _CH), 1)

    def round_fn(t, carry):
        acc, nprev, k0 = carry
        vmin = jnp.full((_QB, _CH), jnp.inf, _f32)
        cidx = jnp.zeros((_QB, _CH), jnp.int32)
        for c in range(_NCH):
            sl = pl.ds(c * _CH, _CH)
            v = keys_ref[:, sl]
            v = jnp.where(lidx + c * _CH == nprev, jnp.inf, v)
            keys_ref[:, sl] = v
            newmin = v < vmin
            cidx = jnp.where(newmin, c, cidx)
            vmin = jnp.minimum(vmin, v)
        m = jnp.min(vmin, axis=1, keepdims=True)
        n_full = cidx * _CH + lidx
        cand = jnp.where(vmin == m, n_full, _BIGI)
        n = jnp.min(cand, axis=1, keepdims=True)
        acc = jnp.where(col == t, jnp.broadcast_to(n, (_QB, _S)), acc)
        k0 = jnp.where(t == 0, m, k0)
        return acc, n, k0

    acc0 = jnp.zeros((_QB, _S), jnp.int32)
    nprev0 = jnp.full((_QB, 1), -1, jnp.int32)
    acc, _, k0 = lax.fori_loop(0, _S, round_fn, (acc0, nprev0, inf))

    no_valid = k0 >= _NOVALID
    idx_ref[...] = jnp.where(no_valid, jnp.broadcast_to(acc[:, 0:1], (_QB, _S)),
                             acc)


def _tc_select(pts, nq, rot9):
    return pl.pallas_call(
        _select_body,
        grid=(_P // _QB,),
        in_specs=[
            pl.BlockSpec((8, _N), lambda i: (0, 0)),
            pl.BlockSpec((_QB, 3), lambda i: (i, 0)),
            pl.BlockSpec((_QB, 9), lambda i: (i, 0)),
        ],
        out_specs=pl.BlockSpec((_QB, _S), lambda i: (i, 0)),
        out_shape=jax.ShapeDtypeStruct((_P, _S), jnp.int32),
        scratch_shapes=[pltpu.VMEM((_QB, _N), _f32)],
    )(pts, nq, rot9)


_NW = 32
_BW = (_P * _S) // _NW
_QW = _BW // _S


def _sc_body(featT, xyzw, idx2, qr, outF, outX,
             idx_v, rows_v, q_v, ox_v, sem):
    wid = lax.axis_index("s") * 2 + lax.axis_index("c")
    base = wid * _BW

    pltpu.sync_copy(idx2.at[pl.ds(wid * 4, 4)], idx_v)
    pltpu.sync_copy(qr.at[pl.ds(wid * _QW, _QW)], q_v)

    copies = []
    for j in range(4):
        copies.append(pltpu.async_copy(
            featT.at[idx_v.at[j]], rows_v.at[pl.ds(j * 128, 128)], sem))
    for c in copies:
        c.wait()
    pltpu.sync_copy(rows_v, outF.at[pl.ds(base, _BW)])

    copies = []
    for j in range(4):
        copies.append(pltpu.async_copy(
            xyzw.at[idx_v.at[j]], rows_v.at[pl.ds(j * 128, 128)], sem))
    for c in copies:
        c.wait()

    li = lax.iota(jnp.int32, 16)
    for pq in range(_QW):
        qrow = q_v[pq, pl.ds(0, 16)]
        cx = qrow[0]
        cy = qrow[1]
        cz = qrow[2]
        r = [qrow[3 + k] for k in range(9)]
        for h in range(2):
            off = pq * _S + h * 16
            gx = jnp.zeros((16,), _f32)
            gy = jnp.zeros((16,), _f32)
            gz = jnp.zeros((16,), _f32)
            for j in range(16):
                row = rows_v[off + j, pl.ds(0, 16)]
                gx = jnp.where(li == j, row[0], gx)
                gy = jnp.where(li == j, row[1], gy)
                gz = jnp.where(li == j, row[2], gz)
            dxb = gx - cx
            dyb = gy - cy
            dzb = gz - cz
            ox_v[0, pl.ds(off, 16)] = (dxb * r[0] + dyb * r[3]) + dzb * r[6]
            ox_v[1, pl.ds(off, 16)] = (dxb * r[1] + dyb * r[4]) + dzb * r[7]
            ox_v[2, pl.ds(off, 16)] = (dxb * r[2] + dyb * r[5]) + dzb * r[8]

    pltpu.sync_copy(ox_v, outX.at[:, pl.ds(base, _BW)])


def _sc_group(featT, xyzw, idx2, qr):
    mesh = plsc.VectorSubcoreMesh(core_axis_name="c", subcore_axis_name="s")
    fn = functools.partial(
        pl.kernel,
        mesh=mesh,
        out_type=[
            jax.ShapeDtypeStruct((_P * _S, _C), _f32),
            jax.ShapeDtypeStruct((3, _P * _S), _f32),
        ],
        scratch_types=[
            pltpu.VMEM((4, 128), jnp.int32),
            pltpu.VMEM((_BW, _C), _f32),
            pltpu.VMEM((_QW, 16), _f32),
            pltpu.VMEM((3, _BW), _f32),
            pltpu.SemaphoreType.DMA,
        ],
    )(_sc_body)
    return fn(featT, xyzw, idx2, qr)


def kernel(xyz, new_xyz, rot, features):
    xyz0 = xyz[0]
    nq = new_xyz[0]
    rot9 = rot[0].reshape(_P, 9)

    pts = jnp.zeros((8, _N), _f32).at[0:3, :].set(jnp.swapaxes(xyz0, 0, 1))
    idx = _tc_select(pts, nq, rot9)

    featT = jnp.swapaxes(features[0], 0, 1)
    idx2 = idx.reshape(128, 128)
    qr = jnp.zeros((_P, 16), _f32)
    qr = qr.at[:, 0:3].set(nq).at[:, 3:12].set(rot9)
    xyzw = jnp.zeros((_N, 128), _f32).at[:, 0:3].set(xyz0)

    outF, outX = _sc_group(featT, xyzw, idx2, qr)
    gx = outX.reshape(1, 3, _P, _S)
    gf = jnp.swapaxes(outF, 0, 1).reshape(1, _C, _P, _S)
    return jnp.concatenate([gx, gf], axis=1)


# speedup vs baseline: 201.9947x; 1.1381x over previous
import functools

import jax
import jax.numpy as jnp
from jax import lax
from jax.experimental import pallas as pl
from jax.experimental.pallas import tpu as pltpu
from jax.experimental.pallas import tpu_sc as plsc

_RADIUS = 0.05
_HMIN = -0.02
_HMAX = 0.04
_S = 32
_N = 16384
_P = 512
_C = 128
_QB = 8
_CH = 1024
_NCH = _N // _CH
_SEP = 0.25
_NOVALID = 0.1
_BIGI = 2 ** 30

_f32 = jnp.float32
_bf = jnp.bfloat16


def _select_body(pts_ref, nq_ref, rot_ref, idx_ref, keys_ref):
    px = pts_ref[0:1, :]
    py = pts_ref[1:2, :]
    pz = pts_ref[2:3, :]
    qx = nq_ref[:, 0:1]
    qy = nq_ref[:, 1:2]
    qz = nq_ref[:, 2:3]

    qb = nq_ref[...].astype(_bf)
    pb = pts_ref[0:3, :].astype(_bf)
    dot = lax.dot_general(qb, pb, (((1,), (0,)), ((), ())),
                          preferred_element_type=_f32)
    q2 = (qx * qx + qy * qy) + qz * qz
    p2 = (px * px + py * py) + pz * pz
    d2 = q2 + p2 - 2.0 * dot

    dx = px - qx
    dy = py - qy
    dz = pz - qz
    stack = jnp.concatenate([dx, dy, dz], axis=0)
    eye8 = (lax.broadcasted_iota(jnp.int32, (_QB, _QB), 0)
            == lax.broadcasted_iota(jnp.int32, (_QB, _QB), 1)).astype(_f32)
    rows = []
    for j in range(3):
        blocks = [rot_ref[:, 3 * j + d:3 * j + d + 1] * eye8 for d in range(3)]
        rows.append(jnp.concatenate(blocks, axis=1))
    rbig = jnp.concatenate(rows, axis=0)
    cyl = lax.dot_general(rbig.astype(_bf), stack.astype(_bf),
                          (((1,), (0,)), ((), ())),
                          preferred_element_type=_f32)
    xc = cyl[0:8]
    yc = cyl[8:16]
    zc = cyl[16:24]

    rad = jnp.sqrt(yc * yc + zc * zc)
    sc = rad + 0.001 * jnp.abs(xc)
    in_cyl = (rad <= _RADIUS) & (xc >= _HMIN) & (xc <= _HMAX)
    keys_ref[...] = jnp.where(in_cyl, sc, _SEP + d2)

    col = lax.broadcasted_iota(jnp.int32, (_QB, _S), 1)
    inf = jnp.full((_QB, 1), jnp.inf, _f32)

    lidx = lax.broadcasted_iota(jnp.int32, (_QB, _CH), 1)

    def round_fn(t, carry):
        acc, nprev, k0 = carry
        vmin = jnp.full((_QB, _CH), jnp.inf, _f32)
        cidx = jnp.zeros((_QB, _CH), jnp.int32)
        for c in range(_NCH):
            sl = pl.ds(c * _CH, _CH)
            v = keys_ref[:, sl]
            v = jnp.where(lidx + c * _CH == nprev, jnp.inf, v)
            keys_ref[:, sl] = v
            newmin = v < vmin
            cidx = jnp.where(newmin, c, cidx)
            vmin = jnp.minimum(vmin, v)
        m = jnp.min(vmin, axis=1, keepdims=True)
        n_full = cidx * _CH + lidx
        cand = jnp.where(vmin == m, n_full, _BIGI)
        n = jnp.min(cand, axis=1, keepdims=True)
        acc = jnp.where(col == t, jnp.broadcast_to(n, (_QB, _S)), acc)
        k0 = jnp.where(t == 0, m, k0)
        return acc, n, k0

    acc0 = jnp.zeros((_QB, _S), jnp.int32)
    nprev0 = jnp.full((_QB, 1), -1, jnp.int32)
    acc, _, k0 = lax.fori_loop(0, _S, round_fn, (acc0, nprev0, inf))

    no_valid = k0 >= _NOVALID
    idx_ref[...] = jnp.where(no_valid, jnp.broadcast_to(acc[:, 0:1], (_QB, _S)),
                             acc)


def _tc_select(pts, nq, rot9):
    return pl.pallas_call(
        _select_body,
        grid=(_P // _QB,),
        in_specs=[
            pl.BlockSpec((8, _N), lambda i: (0, 0)),
            pl.BlockSpec((_QB, 3), lambda i: (i, 0)),
            pl.BlockSpec((_QB, 9), lambda i: (i, 0)),
        ],
        out_specs=pl.BlockSpec((_QB, _S), lambda i: (i, 0)),
        out_shape=jax.ShapeDtypeStruct((_P, _S), jnp.int32),
        scratch_shapes=[pltpu.VMEM((_QB, _N), _f32)],
    )(pts, nq, rot9)


_NW = 32
_BW = (_P * _S) // _NW
_QW = _BW // _S


def _sc_body(featT, xyzw, idx2, qr, outF, outX,
             idx_v, rows_v, q_v, ox_v, sem):
    wid = lax.axis_index("s") * 2 + lax.axis_index("c")
    base = wid * _BW

    pltpu.sync_copy(idx2.at[pl.ds(wid * 4, 4)], idx_v)
    pltpu.sync_copy(qr.at[pl.ds(wid * _QW, _QW)], q_v)

    copies = []
    for j in range(4):
        copies.append(pltpu.async_copy(
            featT.at[idx_v.at[j]], rows_v.at[pl.ds(j * 128, 128)], sem))
    for c in copies:
        c.wait()
    pltpu.sync_copy(rows_v, outF.at[pl.ds(base, _BW)])

    copies = []
    for j in range(4):
        copies.append(pltpu.async_copy(
            xyzw.at[idx_v.at[j]], rows_v.at[pl.ds(j * 128, 128)], sem))
    for c in copies:
        c.wait()

    li = lax.iota(jnp.int32, 16)
    for pq in range(_QW):
        qrow = q_v[pq, pl.ds(0, 16)]
        cx = qrow[0]
        cy = qrow[1]
        cz = qrow[2]
        r = [qrow[3 + k] for k in range(9)]
        for h in range(2):
            off = pq * _S + h * 16
            gx = jnp.zeros((16,), _f32)
            gy = jnp.zeros((16,), _f32)
            gz = jnp.zeros((16,), _f32)
            for j in range(16):
                row = rows_v[off + j, pl.ds(0, 16)]
                gx = jnp.where(li == j, row[0], gx)
                gy = jnp.where(li == j, row[1], gy)
                gz = jnp.where(li == j, row[2], gz)
            dxb = gx - cx
            dyb = gy - cy
            dzb = gz - cz
            ox_v[0, pl.ds(off, 16)] = (dxb * r[0] + dyb * r[3]) + dzb * r[6]
            ox_v[1, pl.ds(off, 16)] = (dxb * r[1] + dyb * r[4]) + dzb * r[7]
            ox_v[2, pl.ds(off, 16)] = (dxb * r[2] + dyb * r[5]) + dzb * r[8]

    pltpu.sync_copy(ox_v, outX.at[:, pl.ds(base, _BW)])


def _sc_group(featT, xyzw, idx2, qr):
    mesh = plsc.VectorSubcoreMesh(core_axis_name="c", subcore_axis_name="s")
    fn = functools.partial(
        pl.kernel,
        mesh=mesh,
        out_type=[
            jax.ShapeDtypeStruct((_P * _S, _C), _f32),
            jax.ShapeDtypeStruct((3, _P * _S), _f32),
        ],
        scratch_types=[
            pltpu.VMEM((4, 128), jnp.int32),
            pltpu.VMEM((_BW, _C), _f32),
            pltpu.VMEM((_QW, 16), _f32),
            pltpu.VMEM((3, _BW), _f32),
            pltpu.SemaphoreType.DMA,
        ],
    )(_sc_body)
    return fn(featT, xyzw, idx2, qr)


def kernel(xyz, new_xyz, rot, features):
    xyz0 = xyz[0]
    nq = new_xyz[0]
    rot9 = rot[0].reshape(_P, 9)

    pts = jnp.zeros((8, _N), _f32).at[0:3, :].set(jnp.swapaxes(xyz0, 0, 1))
    idx = _tc_select(pts, nq, rot9)

    featT = jnp.swapaxes(features[0], 0, 1)
    idx2 = idx.reshape(128, 128)
    qr = jnp.zeros((_P, 16), _f32)
    qr = qr.at[:, 0:3].set(nq).at[:, 3:12].set(rot9)
    xyzw = jnp.zeros((_N, 128), _f32).at[:, 0:3].set(xyz0)

    outF, outX = _sc_group(featT, xyzw, idx2, qr)
    gx = outX.reshape(1, 3, _P, _S)
    gf = jnp.swapaxes(outF, 0, 1).reshape(1, _C, _P, _S)
    return jnp.concatenate([gx, gf], axis=1)
